# Initial kernel scaffold; baseline (speedup 1.0000x reference)
import jax
import jax.numpy as jnp
from jax.experimental import pallas as pl


def kernel(x, table, W, b):
    raise NotImplementedError("write your pallas kernel here")


# speedup vs baseline: 69.7546x; 69.7546x over previous
import jax
import jax.numpy as jnp
from jax import lax
from jax.experimental import pallas as pl
from jax.experimental.pallas import tpu as pltpu
from jax.experimental.pallas import tpu_sc as plsc

_B = 16384
_P = 26
_V = 100000
_NC = 2
_NS = 16
_L = 16
_NW = _NC * _NS
_ROWS_W = _B // _NW
_GROUPS = _ROWS_W // _L
_VPAD = 100096
_WBN = 512


def _sc_body(x_hbm, table_hbm, wb_hbm, out_hbm, table_v, x_v, wb_v, out_v):
    wid = lax.axis_index("s") * _NC + lax.axis_index("c")
    base = wid * _ROWS_W

    pltpu.sync_copy(table_hbm, table_v)
    pltpu.sync_copy(x_hbm.at[pl.ds(base * _P, _ROWS_W * _P)], x_v)
    pltpu.sync_copy(wb_hbm, wb_v)

    iota_p = lax.iota(jnp.int32, _L) * _P
    wvecs = [wb_v[pl.ds(p * _L, _L)] for p in range(_P)]
    bvec = wb_v[pl.ds(_P * _L, _L)]

    @pl.loop(0, _GROUPS)
    def _(g):
        gbase = g * (_L * _P)
        acc = bvec
        for p in range(_P):
            addr = iota_p + (gbase + p)
            idx = plsc.load_gather(x_v, [addr])
            vals = plsc.load_gather(table_v, [idx])
            acc = acc + vals * wvecs[p]
        out_v[pl.ds(g * _L, _L)] = acc

    pltpu.sync_copy(out_v, out_hbm.at[pl.ds(base, _ROWS_W)])


@jax.jit
def kernel(x, table, W, b):
    xf = x.reshape(-1).astype(jnp.int32)
    tf = jnp.pad(table.reshape(-1), (0, _VPAD - _V))
    wb = jnp.concatenate([W.reshape(-1), b.reshape(-1)])
    wb = jnp.pad(jnp.repeat(wb, _L), (0, _WBN - (_P + 1) * _L))
    out = pl.kernel(
        _sc_body,
        out_type=jax.ShapeDtypeStruct((_B,), jnp.float32),
        mesh=plsc.VectorSubcoreMesh(
            core_axis_name="c", subcore_axis_name="s",
            num_cores=_NC, num_subcores=_NS,
        ),
        compiler_params=pltpu.CompilerParams(needs_layout_passes=False),
        scratch_types=[
            pltpu.VMEM((_VPAD,), jnp.float32),
            pltpu.VMEM((_ROWS_W * _P,), jnp.int32),
            pltpu.VMEM((_WBN,), jnp.float32),
            pltpu.VMEM((_ROWS_W,), jnp.float32),
        ],
    )(xf, tf, wb)
    return out.reshape(_B, 1)


# speedup vs baseline: 124.4807x; 1.7846x over previous
import jax
import jax.numpy as jnp
from jax import lax
from jax.experimental import pallas as pl
from jax.experimental.pallas import tpu as pltpu
from jax.experimental.pallas import tpu_sc as plsc

_B = 16384
_P = 26
_V = 100000
_NC = 2
_NS = 16
_L = 16
_ACT = 8
_NW = _NC * _ACT
_ROWS_W = _B // _NW
_GROUPS = _ROWS_W // _L
_WBN = 512


def _sc_body(xt_hbm, table_hbm, wb_hbm, out_hbm,
             table_v, x_v, wb_v, out_v, sem_t, sem_x, sem_w):
    s = lax.axis_index("s")

    @pl.when(s < _ACT)
    def _():
        wid = s * _NC + lax.axis_index("c")
        base = wid * _ROWS_W

        tcopy = pltpu.async_copy(table_hbm.at[0], table_v, sem_t)
        xcopies = [
            pltpu.async_copy(
                xt_hbm.at[p, pl.ds(base, _ROWS_W)],
                x_v.at[pl.ds(p * _ROWS_W, _ROWS_W)], sem_x,
            )
            for p in range(_P)
        ]
        wcopy = pltpu.async_copy(wb_hbm, wb_v, sem_w)

        wcopy.wait()
        wvecs = [wb_v[pl.ds(p * _L, _L)] for p in range(_P)]
        bvec = wb_v[pl.ds(_P * _L, _L)]
        for c in xcopies:
            c.wait()
        tcopy.wait()

        @pl.loop(0, _GROUPS, unroll=2)
        def _(g):
            col = g * _L
            accs = [bvec, None, None, None]
            for p in range(_P):
                idx = x_v[pl.ds(p * _ROWS_W + col, _L)]
                vals = plsc.load_gather(table_v, [idx])
                term = vals * wvecs[p]
                k = p % 4
                accs[k] = term if accs[k] is None else accs[k] + term
            out_v[pl.ds(col, _L)] = (accs[0] + accs[1]) + (accs[2] + accs[3])

        pltpu.sync_copy(out_v, out_hbm.at[pl.ds(base, _ROWS_W)])


@jax.jit
def kernel(x, table, W, b):
    xt = x.T
    tt = table.T
    wb = jnp.concatenate([W.reshape(-1), b.reshape(-1)])
    wb = jnp.pad(jnp.repeat(wb, _L), (0, _WBN - (_P + 1) * _L))
    out = pl.kernel(
        _sc_body,
        out_type=jax.ShapeDtypeStruct((_B,), jnp.float32),
        mesh=plsc.VectorSubcoreMesh(
            core_axis_name="c", subcore_axis_name="s",
            num_cores=_NC, num_subcores=_NS,
        ),
        compiler_params=pltpu.CompilerParams(needs_layout_passes=False),
        scratch_types=[
            pltpu.VMEM((_V,), jnp.float32),
            pltpu.VMEM((_P * _ROWS_W,), jnp.int32),
            pltpu.VMEM((_WBN,), jnp.float32),
            pltpu.VMEM((_ROWS_W,), jnp.float32),
            pltpu.SemaphoreType.DMA,
            pltpu.SemaphoreType.DMA,
            pltpu.SemaphoreType.DMA,
        ],
    )(xt, tt, wb)
    return out.reshape(_B, 1)


# speedup vs baseline: 124.6607x; 1.0014x over previous
import jax
import jax.numpy as jnp
from jax import lax
from jax.experimental import pallas as pl
from jax.experimental.pallas import tpu as pltpu
from jax.experimental.pallas import tpu_sc as plsc

_B = 16384
_P = 26
_V = 100000
_NC = 2
_NS = 16
_L = 16
_ACT = 8
_NW = _NC * _ACT
_ROWS_W = _B // _NW
_GROUPS = _ROWS_W // _L
_WBN = 512


def _sc_body(xt_hbm, table_hbm, wb_hbm, out_hbm,
             table_v, x_v, wb_v, out_v, sem_t, sem_x, sem_w):
    s = lax.axis_index("s")

    @pl.when(s < _ACT)
    def _():
        wid = s * _NC + lax.axis_index("c")
        base = wid * _ROWS_W

        tchunk = 25088
        toffs = [0, tchunk, 2 * tchunk, 3 * tchunk]
        tcopies = [
            pltpu.async_copy(
                table_hbm.at[0, pl.ds(o, min(tchunk, _V - o))],
                table_v.at[pl.ds(o, min(tchunk, _V - o))], sem_t,
            )
            for o in toffs
        ]
        xcopies = [
            pltpu.async_copy(
                xt_hbm.at[p, pl.ds(base, _ROWS_W)],
                x_v.at[pl.ds(p * _ROWS_W, _ROWS_W)], sem_x,
            )
            for p in range(_P)
        ]
        wcopy = pltpu.async_copy(wb_hbm, wb_v, sem_w)

        wcopy.wait()
        wvecs = [
            plsc.load_gather(wb_v, [jnp.full((_L,), p, jnp.int32)])
            for p in range(_P)
        ]
        bvec = plsc.load_gather(wb_v, [jnp.full((_L,), _P, jnp.int32)])
        for c in xcopies:
            c.wait()
        for c in tcopies:
            c.wait()

        @pl.loop(0, _GROUPS, unroll=4)
        def _(g):
            col = g * _L
            accs = [bvec, None, None, None]
            for p in range(_P):
                idx = x_v[pl.ds(p * _ROWS_W + col, _L)]
                vals = plsc.load_gather(table_v, [idx])
                term = vals * wvecs[p]
                k = p % 4
                accs[k] = term if accs[k] is None else accs[k] + term
            out_v[pl.ds(col, _L)] = (accs[0] + accs[1]) + (accs[2] + accs[3])

        pltpu.sync_copy(out_v, out_hbm.at[pl.ds(base, _ROWS_W)])


@jax.jit
def kernel(x, table, W, b):
    xt = x.T
    tt = table.T
    wb = jnp.pad(jnp.concatenate([W.reshape(-1), b.reshape(-1)]), (0, 5))
    out = pl.kernel(
        _sc_body,
        out_type=jax.ShapeDtypeStruct((_B,), jnp.float32),
        mesh=plsc.VectorSubcoreMesh(
            core_axis_name="c", subcore_axis_name="s",
            num_cores=_NC, num_subcores=_NS,
        ),
        compiler_params=pltpu.CompilerParams(
            needs_layout_passes=False, skip_device_barrier=True,
        ),
        scratch_types=[
            pltpu.VMEM((_V,), jnp.float32),
            pltpu.VMEM((_P * _ROWS_W,), jnp.int32),
            pltpu.VMEM((32,), jnp.float32),
            pltpu.VMEM((_ROWS_W,), jnp.float32),
            pltpu.SemaphoreType.DMA,
            pltpu.SemaphoreType.DMA,
            pltpu.SemaphoreType.DMA,
        ],
    )(xt, tt, wb)
    return out.reshape(_B, 1)


# speedup vs baseline: 124.8699x; 1.0017x over previous
import jax
import jax.numpy as jnp
from jax import lax
from jax.experimental import pallas as pl
from jax.experimental.pallas import tpu as pltpu
from jax.experimental.pallas import tpu_sc as plsc

_B = 16384
_P = 26
_V = 100000
_NC = 2
_NS = 16
_L = 16
_ACT = 8
_NW = _NC * _ACT
_ROWS_W = _B // _NW
_GROUPS = _ROWS_W // _L
_WBN = 512


def _sc_body(xt_hbm, table_hbm, wb_hbm, out_hbm,
             table_v, x_v, wb_v, out_v, sem_t, sem_x, sem_w):
    s = lax.axis_index("s")

    @pl.when(s < _ACT)
    def _():
        wid = s * _NC + lax.axis_index("c")
        base = wid * _ROWS_W

        tchunk = 25088
        toffs = [0, tchunk, 2 * tchunk, 3 * tchunk]
        tcopies = [
            pltpu.async_copy(
                table_hbm.at[0, pl.ds(o, min(tchunk, _V - o))],
                table_v.at[pl.ds(o, min(tchunk, _V - o))], sem_t,
            )
            for o in toffs
        ]
        xcopies = [
            pltpu.async_copy(
                xt_hbm.at[p, pl.ds(base, _ROWS_W)],
                x_v.at[pl.ds(p * _ROWS_W, _ROWS_W)], sem_x,
            )
            for p in range(_P)
        ]
        wcopy = pltpu.async_copy(wb_hbm, wb_v, sem_w)

        wcopy.wait()
        wlo = wb_v[pl.ds(0, _L)]
        whi = wb_v[pl.ds(_L, _L)]
        def _splat(vec, lane):
            return jnp.take_along_axis(
                vec, jnp.full((_L,), lane, jnp.int32), axis=0
            )
        wvecs = [
            _splat(wlo, p) if p < _L else _splat(whi, p - _L)
            for p in range(_P)
        ]
        bvec = _splat(whi, _P - _L)
        for c in xcopies:
            c.wait()
        for c in tcopies:
            c.wait()

        @pl.loop(0, _GROUPS, unroll=4)
        def _(g):
            col = g * _L
            accs = [bvec, None, None, None]
            for p in range(_P):
                idx = x_v[pl.ds(p * _ROWS_W + col, _L)]
                vals = plsc.load_gather(table_v, [idx])
                term = vals * wvecs[p]
                k = p % 4
                accs[k] = term if accs[k] is None else accs[k] + term
            out_v[pl.ds(col, _L)] = (accs[0] + accs[1]) + (accs[2] + accs[3])

        pltpu.sync_copy(out_v, out_hbm.at[pl.ds(base, _ROWS_W)])


@jax.jit
def kernel(x, table, W, b):
    xt = x.T
    tt = table.T
    wb = jnp.pad(jnp.concatenate([W.reshape(-1), b.reshape(-1)]), (0, 5))
    out = pl.kernel(
        _sc_body,
        out_type=jax.ShapeDtypeStruct((_B,), jnp.float32),
        mesh=plsc.VectorSubcoreMesh(
            core_axis_name="c", subcore_axis_name="s",
            num_cores=_NC, num_subcores=_NS,
        ),
        compiler_params=pltpu.CompilerParams(needs_layout_passes=False),
        scratch_types=[
            pltpu.VMEM((_V,), jnp.float32),
            pltpu.VMEM((_P * _ROWS_W,), jnp.int32),
            pltpu.VMEM((32,), jnp.float32),
            pltpu.VMEM((_ROWS_W,), jnp.float32),
            pltpu.SemaphoreType.DMA,
            pltpu.SemaphoreType.DMA,
            pltpu.SemaphoreType.DMA,
        ],
    )(xt, tt, wb)
    return out.reshape(_B, 1)
